# Initial kernel scaffold; baseline (speedup 1.0000x reference)
import jax
import jax.numpy as jnp
from jax.experimental import pallas as pl


def kernel(x, edge_index, edge_weight, W1, b1, W2, b2):
    raise NotImplementedError("write your pallas kernel here")


# speedup vs baseline: 4.4486x; 4.4486x over previous
import functools

import jax
import jax.numpy as jnp
from jax import lax
from jax.experimental import pallas as pl
from jax.experimental.pallas import tpu as pltpu
from jax.experimental.pallas import tpu_sc as plsc

_N = 10000
_E = 320000
_F = 128
_H = 16
_O = 7

_NC = 2
_NS = 16
_NW = _NC * _NS
_EPW = _E // _NW
_CH = 80
_NCHUNK = _EPW // _CH
_NP = 10240
_RPS = _NP // _NS


def _mm1_body(x_ref, w_ref, o_ref):
    o_ref[:] = jnp.dot(x_ref[:], w_ref[:], preferred_element_type=jnp.float32)


def _mm1(x, W1):
    return pl.pallas_call(
        _mm1_body,
        out_shape=jax.ShapeDtypeStruct((_N, _H), jnp.float32),
    )(x, W1)


def _mid_body(p_ref, b1_ref, w2_ref, o_ref):
    s = p_ref[0, :_N] + p_ref[1, :_N] + b1_ref[:]
    o_ref[:] = jnp.dot(s, w2_ref[:], preferred_element_type=jnp.float32)


def _mid(p1, b1_2d, W2p):
    return pl.pallas_call(
        _mid_body,
        out_shape=jax.ShapeDtypeStruct((_N, _H), jnp.float32),
    )(p1, b1_2d, W2p)


def _fin_body(p_ref, b2_ref, o_ref):
    a = p_ref[0, :_N] + p_ref[1, :_N]
    o_ref[:] = a[:, :_O] + b2_ref[:]


def _fin(p2, b2_2d):
    return pl.pallas_call(
        _fin_body,
        out_shape=jax.ShapeDtypeStruct((_N, _O), jnp.float32),
    )(p2, b2_2d)


def _sc_agg(h, src, dst, w):
    mesh = plsc.VectorSubcoreMesh(core_axis_name="c", subcore_axis_name="s")

    @functools.partial(
        pl.kernel,
        out_type=jax.ShapeDtypeStruct((_NC, _NP, _H), jnp.float32),
        mesh=mesh,
        scratch_types=[
            pltpu.VMEM_SHARED((_NP, _H), jnp.float32),
            pltpu.VMEM((_CH,), jnp.int32),
            pltpu.VMEM((_CH,), jnp.int32),
            pltpu.VMEM((_CH,), jnp.float32),
            pltpu.VMEM((_CH, _H), jnp.float32),
            pltpu.VMEM((_RPS, _H), jnp.float32),
            pltpu.SemaphoreType.DMA,
        ],
        compiler_params=pltpu.CompilerParams(
            needs_layout_passes=False, use_tc_tiling_on_sc=False),
    )
    def k(h_hbm, src_hbm, dst_hbm, w_hbm, out_hbm,
          acc, src_v, dst_v, w_v, rows_v, zrow, sem):
        cid = lax.axis_index("c")
        sid = lax.axis_index("s")
        wid = cid * _NS + sid

        def zbody(i, carry):
            zrow[i, :] = jnp.zeros((_H,), jnp.float32)
            return carry
        lax.fori_loop(0, _RPS, zbody, 0)
        pltpu.sync_copy(zrow, acc.at[pl.ds(sid * _RPS, _RPS)])
        plsc.subcore_barrier()

        ebase = wid * _EPW

        def chunk(j, carry):
            base = pl.multiple_of(ebase + j * _CH, 8)
            pltpu.sync_copy(src_hbm.at[pl.ds(base, _CH)], src_v)
            pltpu.sync_copy(dst_hbm.at[pl.ds(base, _CH)], dst_v)
            pltpu.sync_copy(w_hbm.at[pl.ds(base, _CH)], w_v)
            pltpu.async_copy(h_hbm.at[src_v], rows_v, sem).wait()

            def ebody(e, c2):
                wbc = plsc.load_gather(w_v, [jnp.full((_H,), e, jnp.int32)])
                rows_v[e, :] = rows_v[e, :] * wbc
                return c2
            lax.fori_loop(0, _CH, ebody, 0)

            pltpu.sync_copy(rows_v, acc.at[dst_v], add=True)
            return carry
        lax.fori_loop(0, _NCHUNK, chunk, 0)

        plsc.subcore_barrier()
        pltpu.sync_copy(acc.at[pl.ds(sid * _RPS, _RPS)],
                        out_hbm.at[cid, pl.ds(sid * _RPS, _RPS)])

    return k(h, src, dst, w)


def kernel(x, edge_index, edge_weight, W1, b1, W2, b2):
    src = edge_index[0]
    dst = edge_index[1]
    W2p = jnp.pad(W2, ((0, 0), (0, _H - _O)))
    b1_2d = b1.reshape(1, _H)
    b2_2d = b2.reshape(1, _O)

    h1 = _mm1(x, W1)
    p1 = _sc_agg(h1, src, dst, edge_weight)
    h2 = _mid(p1, b1_2d, W2p)
    p2 = _sc_agg(h2, src, dst, edge_weight)
    return _fin(p2, b2_2d)


# speedup vs baseline: 7.4047x; 1.6645x over previous
import functools

import jax
import jax.numpy as jnp
from jax import lax
from jax.experimental import pallas as pl
from jax.experimental.pallas import tpu as pltpu
from jax.experimental.pallas import tpu_sc as plsc

_N = 10000
_E = 320000
_F = 128
_H = 16
_O = 7

_NP = 10240
_EP = 327680
_ER = _EP // 128
_NG = 8
_RPG = _ER // _NG
_FPT = 4
_BR = 8
_NBLK = _RPG // _BR
_FLAT = _FPT * _NP


def _mm1_body(x_ref, w_ref, o_ref):
    hT = lax.dot_general(w_ref[:], x_ref[:], (((0,), (1,)), ((), ())),
                         preferred_element_type=jnp.float32)
    o_ref[:] = jnp.concatenate(
        [hT, jnp.zeros((_H, _NP - _N), jnp.float32)], axis=1)


def _mm1(x, W1):
    return pl.pallas_call(
        _mm1_body,
        out_shape=jax.ShapeDtypeStruct((_H, _NP), jnp.float32),
    )(x, W1)


def _mid_body(p_ref, b1_ref, w2_ref, o_ref):
    aggT = jnp.sum(p_ref[:], axis=0) + b1_ref[:]
    o_ref[:] = lax.dot_general(w2_ref[:], aggT, (((0,), (0,)), ((), ())),
                               preferred_element_type=jnp.float32)


def _mid(p1, b1_col, W2p):
    return pl.pallas_call(
        _mid_body,
        out_shape=jax.ShapeDtypeStruct((_H, _NP), jnp.float32),
    )(p1, b1_col, W2p)


def _fin_body(p_ref, b2_ref, eye_ref, o_ref):
    aggT = jnp.sum(p_ref[:], axis=0)
    agg = lax.dot_general(aggT, eye_ref[:], (((0,), (0,)), ((), ())),
                          preferred_element_type=jnp.float32)
    o_ref[:] = agg[:_N, :_O] + b2_ref[:]


def _fin(p2, b2_2d):
    return pl.pallas_call(
        _fin_body,
        out_shape=jax.ShapeDtypeStruct((_N, _O), jnp.float32),
    )(p2, b2_2d, jnp.eye(_H, dtype=jnp.float32))


def _sc_agg(hT_flat, src2d, dst2d, w2d):
    mesh = plsc.VectorSubcoreMesh(core_axis_name="c", subcore_axis_name="s")

    @functools.partial(
        pl.kernel,
        out_type=jax.ShapeDtypeStruct((_NG, _H * _NP), jnp.float32),
        mesh=mesh,
        scratch_types=[
            pltpu.VMEM((_FLAT,), jnp.float32),
            pltpu.VMEM((_FLAT,), jnp.float32),
            pltpu.VMEM((_BR, 128), jnp.int32),
            pltpu.VMEM((_BR, 128), jnp.int32),
            pltpu.VMEM((_BR, 128), jnp.float32),
        ],
        compiler_params=pltpu.CompilerParams(
            needs_layout_passes=False, use_tc_tiling_on_sc=False),
    )
    def k(hT_hbm, src_hbm, dst_hbm, w_hbm, out_hbm,
          hT_v, acc_v, src_b, dst_b, w_b):
        cid = lax.axis_index("c")
        sid = lax.axis_index("s")
        wid = cid * 16 + sid
        grp = wid // _FPT
        fbase = (wid % _FPT) * _FPT

        pltpu.sync_copy(hT_hbm.at[pl.ds(fbase * _NP, _FLAT)], hT_v)

        zero16 = jnp.zeros((16,), jnp.float32)

        def zb(i, c):
            acc_v[pl.ds(i * 16, 16)] = zero16
            return c
        lax.fori_loop(0, _FLAT // 16, zb, 0)

        row0g = grp * _RPG

        def blk(j, carry):
            r0 = pl.multiple_of(row0g + j * _BR, 8)
            pltpu.sync_copy(src_hbm.at[pl.ds(r0, _BR)], src_b)
            pltpu.sync_copy(dst_hbm.at[pl.ds(r0, _BR)], dst_b)
            pltpu.sync_copy(w_hbm.at[pl.ds(r0, _BR)], w_b)

            def brow(b, c2):
                for t in range(8):
                    src16 = src_b[b, pl.ds(t * 16, 16)]
                    dst16 = dst_b[b, pl.ds(t * 16, 16)]
                    w16 = w_b[b, pl.ds(t * 16, 16)]
                    for fi in range(_FPT):
                        off = jnp.int32(fi * _NP)
                        vals = plsc.load_gather(hT_v, [src16 + off]) * w16
                        plsc.addupdate_scatter(acc_v, [dst16 + off], vals)
                return c2
            lax.fori_loop(0, _BR, brow, 0)
            return carry
        lax.fori_loop(0, _NBLK, blk, 0)

        pltpu.sync_copy(acc_v, out_hbm.at[grp, pl.ds(fbase * _NP, _FLAT)])

    return k(hT_flat, src2d, dst2d, w2d)


def kernel(x, edge_index, edge_weight, W1, b1, W2, b2):
    pad = _EP - _E
    src2d = jnp.concatenate(
        [edge_index[0], jnp.zeros((pad,), jnp.int32)]).reshape(_ER, 128)
    dst2d = jnp.concatenate(
        [edge_index[1], jnp.zeros((pad,), jnp.int32)]).reshape(_ER, 128)
    w2d = jnp.concatenate(
        [edge_weight, jnp.zeros((pad,), jnp.float32)]).reshape(_ER, 128)
    W2p = jnp.pad(W2, ((0, 0), (0, _H - _O)))
    b1_col = b1.reshape(_H, 1)
    b2_2d = b2.reshape(1, _O)

    h1T = _mm1(x, W1)
    p1 = _sc_agg(h1T.reshape(_H * _NP), src2d, dst2d, w2d)
    h2T = _mid(p1.reshape(_NG, _H, _NP), b1_col, W2p)
    p2 = _sc_agg(h2T.reshape(_H * _NP), src2d, dst2d, w2d)
    return _fin(p2.reshape(_NG, _H, _NP), b2_2d)
